# baseline (device time: 59940 ns/iter reference)
import jax
import jax.numpy as jnp
from jax import lax
from jax.experimental import pallas as pl
from jax.experimental.pallas import tpu as pltpu


def kernel(Q, K, V):
    b, s, h, d = Q.shape
    bh = b * h

    Qt = jnp.transpose(Q, (0, 2, 1, 3)).reshape(bh, s, d)
    Kt = jnp.transpose(K, (0, 2, 1, 3)).reshape(bh, s, d)
    Vt = jnp.transpose(V, (0, 2, 1, 3)).reshape(bh, s, d)
    KVt = jnp.concatenate([Kt, Vt], axis=0)

    scale = d ** -0.5

    def body(q_ref, kv_ref, o_ref, kvr_ref, m_ref, r_ref, send_sem, recv_sem):
        my_x = lax.axis_index("x")
        my_y = lax.axis_index("y")
        peer = (1 - my_x, my_y)

        barrier_sem = pltpu.get_barrier_semaphore()
        pl.semaphore_signal(
            barrier_sem, inc=1, device_id=peer,
            device_id_type=pl.DeviceIdType.MESH,
        )
        pl.semaphore_wait(barrier_sem, 1)

        rdma = pltpu.make_async_remote_copy(
            src_ref=kv_ref, dst_ref=kvr_ref,
            send_sem=send_sem, recv_sem=recv_sem,
            device_id=peer, device_id_type=pl.DeviceIdType.MESH,
        )
        rdma.start()

        for i in range(bh):
            q = q_ref[i] * scale
            s1 = lax.dot_general(
                q, kv_ref[i], (((1,), (1,)), ((), ())),
                preferred_element_type=jnp.float32)
            m1 = jnp.max(s1, axis=-1, keepdims=True)
            p1 = jnp.exp(s1 - m1)
            r1 = jnp.sum(p1, axis=-1, keepdims=True)
            o1 = lax.dot_general(
                p1, kv_ref[bh + i], (((1,), (0,)), ((), ())),
                preferred_element_type=jnp.float32)
            o_ref[i] = o1
            m_ref[i] = m1
            r_ref[i] = r1

        rdma.wait()

        for i in range(bh):
            q = q_ref[i] * scale
            s2 = lax.dot_general(
                q, kvr_ref[i], (((1,), (1,)), ((), ())),
                preferred_element_type=jnp.float32)
            m1 = m_ref[i]
            m2 = jnp.max(s2, axis=-1, keepdims=True)
            m = jnp.maximum(m1, m2)
            p2 = jnp.exp(s2 - m)
            alpha = jnp.exp(m1 - m)
            o2 = lax.dot_general(
                p2, kvr_ref[bh + i], (((1,), (0,)), ((), ())),
                preferred_element_type=jnp.float32)
            denom = r_ref[i] * alpha + jnp.sum(p2, axis=-1, keepdims=True)
            o_ref[i] = (o_ref[i] * alpha + o2) / denom

    out = pl.pallas_call(
        body,
        out_shape=jax.ShapeDtypeStruct((bh, s, d), jnp.float32),
        in_specs=[pl.BlockSpec(memory_space=pltpu.VMEM)] * 2,
        out_specs=pl.BlockSpec(memory_space=pltpu.VMEM),
        scratch_shapes=[
            pltpu.VMEM((2 * bh, s, d), jnp.float32),
            pltpu.VMEM((bh, s, 1), jnp.float32),
            pltpu.VMEM((bh, s, 1), jnp.float32),
            pltpu.SemaphoreType.DMA,
            pltpu.SemaphoreType.DMA,
        ],
        compiler_params=pltpu.CompilerParams(collective_id=0),
    )(Qt, KVt)

    return out.reshape(b, h, s, d).transpose(0, 2, 1, 3)
